# baseline (speedup 1.0000x reference)
import functools

import jax
import jax.numpy as jnp
from jax import lax
from jax.experimental import pallas as pl
from jax.experimental.pallas import tpu as pltpu
from jax.experimental.pallas import tpu_sc as plsc

D = 256
KNN = 20
SAMPLE = 40
DEG = KNN + SAMPLE
N_REC = 1024
N_LIG = 256
N_TOT = N_REC + N_LIG
E_TOT = N_TOT * DEG

_NEG_INF = float("-inf")
_POS_INF = float("inf")


def _sig(x):
    return 1.0 / (1.0 + jnp.exp(-x))


def _select_body(N, W, B, pts_ref, ptsT_ref, gp_ref, idx_ref, rad_ref):
    px = pts_ref[:, 0:1]
    py = pts_ref[:, 1:2]
    pz = pts_ref[:, 2:3]
    qx = ptsT_ref[0:1, :]
    qy = ptsT_ref[1:2, :]
    qz = ptsT_ref[2:3, :]
    dx = px - qx
    dy = py - qy
    dz = pz - qz
    d2 = (dx * dx + dy * dy) + dz * dz
    dist = jnp.sqrt(d2 + 1e-12)
    iota = lax.broadcasted_iota(jnp.int32, (B, N), 1)

    work = dist
    isknn = jnp.zeros((B, N), jnp.bool_)
    cnt = jnp.zeros((B, N), jnp.float32)
    for k in range(KNN):
        mv = jnp.min(work, axis=1, keepdims=True)
        idxk = jnp.min(jnp.where(work == mv, iota, N), axis=1, keepdims=True)
        sel = iota == idxk
        rad_ref[:, k:k + 1] = mv * mv - 1e-12
        idx_ref[:, k:k + 1] = idxk
        work = jnp.where(sel, _POS_INF, work)
        isknn = jnp.logical_or(isknn, sel)
        cnt = cnt + (iota >= idxk).astype(jnp.float32)

    base = jnp.log(1.0 / (dist * dist * dist))
    gexp = jnp.zeros((B, N), jnp.float32)
    for cc in range(KNN + 1):
        sl = gp_ref[:, KNN + 1 - cc: KNN + 1 - cc + N]
        gexp = jnp.where(cnt == cc, sl, gexp)
    score = jnp.where(isknn, _NEG_INF, base + gexp)

    for k in range(SAMPLE):
        mv = jnp.max(score, axis=1, keepdims=True)
        idxk = jnp.min(jnp.where(score == mv, iota, N), axis=1, keepdims=True)
        sel = iota == idxk
        rad_ref[:, KNN + k:KNN + k + 1] = jnp.sum(
            jnp.where(sel, d2, 0.0), axis=1, keepdims=True)
        idx_ref[:, KNN + k:KNN + k + 1] = idxk
        score = jnp.where(sel, _NEG_INF, score)

    idx_ref[:, DEG:] = jnp.zeros((B, 64 - DEG), jnp.int32)
    rad_ref[:, DEG:] = jnp.zeros((B, 64 - DEG), jnp.float32)


def _select(pts, ptsT, gp, N, W, B=128):
    grid = N // B
    fn = pl.pallas_call(
        functools.partial(_select_body, N, W, B),
        grid=(grid,),
        in_specs=[
            pl.BlockSpec((B, 8), lambda i: (i, 0)),
            pl.BlockSpec((8, N), lambda i: (0, 0)),
            pl.BlockSpec((B, W), lambda i: (i, 0)),
        ],
        out_specs=[
            pl.BlockSpec((B, 64), lambda i: (i, 0)),
            pl.BlockSpec((B, 64), lambda i: (i, 0)),
        ],
        out_shape=[
            jax.ShapeDtypeStruct((N, 64), jnp.int32),
            jax.ShapeDtypeStruct((N, 64), jnp.float32),
        ],
    )
    return fn(pts, ptsT, gp)


def _proj_body(h_ref, wst_ref, wdt_ref, bias_ref, hs_ref, hd_ref):
    hh = h_ref[...]
    hs_ref[...] = jnp.dot(hh, wst_ref[...],
                          preferred_element_type=jnp.float32) + bias_ref[0:1, :]
    hd_ref[...] = jnp.dot(hh, wdt_ref[...],
                          preferred_element_type=jnp.float32).astype(jnp.bfloat16)


def _proj(h, wst, wdt, biasS):
    fn = pl.pallas_call(
        _proj_body,
        out_shape=[
            jax.ShapeDtypeStruct((N_TOT, D), jnp.float32),
            jax.ShapeDtypeStruct((N_TOT, D), jnp.bfloat16),
        ],
    )
    return fn(h, wst, wdt, biasS)


_SC_WORKERS = 32
_SC_CHUNK = 120
_SC_PER_W = E_TOT // _SC_WORKERS
_SC_NCHUNK = _SC_PER_W // _SC_CHUNK


def _sc_gather_body(table_hbm, idx_hbm, out_hbm, idx_v, rows_v, sem):
    wid = lax.axis_index("s") * 2 + lax.axis_index("c")
    base = wid * _SC_PER_W

    def body(c, carry):
        off = base + c * _SC_CHUNK
        pltpu.sync_copy(idx_hbm.at[pl.ds(off, _SC_CHUNK)], idx_v)
        pltpu.async_copy(table_hbm.at[idx_v], rows_v, sem).wait()
        pltpu.sync_copy(rows_v, out_hbm.at[pl.ds(off, _SC_CHUNK)])
        return carry

    lax.fori_loop(0, _SC_NCHUNK, body, 0)


def _sc_gather(table, idx):
    mesh = plsc.VectorSubcoreMesh(core_axis_name="c", subcore_axis_name="s")
    fn = pl.kernel(
        _sc_gather_body,
        out_type=jax.ShapeDtypeStruct((E_TOT, D // 2), jnp.float32),
        mesh=mesh,
        scratch_types=[
            pltpu.VMEM((_SC_CHUNK,), jnp.int32),
            pltpu.VMEM((_SC_CHUNK, D // 2), jnp.float32),
            pltpu.SemaphoreType.DMA,
        ],
    )
    return fn(table, idx)


_NB = 16
_EB = _NB * DEG


def _edge_body(hdg_ref, hs_ref, h_ref, rad_ref, we2t_ref,
               wn1ht_ref, wn1at_ref, wn2t_ref, vec_ref, out_ref):
    f32 = jnp.float32
    row_node = lax.broadcasted_iota(jnp.int32, (_EB, _NB), 0) // DEG
    col16 = lax.broadcasted_iota(jnp.int32, (_EB, _NB), 1)
    A = (row_node == col16).astype(f32)
    rowm = lax.broadcasted_iota(jnp.int32, (_EB, DEG), 0) % DEG
    colm = lax.broadcasted_iota(jnp.int32, (_EB, DEG), 1)
    C = (rowm == colm).astype(f32)
    srow = lax.broadcasted_iota(jnp.int32, (_NB, _EB), 0)
    scol = lax.broadcasted_iota(jnp.int32, (_NB, _EB), 1) // DEG
    S = (srow == scol).astype(f32)

    be2 = vec_ref[0:1, :]
    bn1 = vec_ref[1:2, :]
    bn2 = vec_ref[2:3, :]
    ba = vec_ref[3:4, 0:1]
    wr = vec_ref[4:5, :]
    wa = vec_ref[5:6, :]

    radB = rad_ref[:, :DEG]
    t1 = jnp.dot(A, radB, preferred_element_type=f32)
    rflat = jnp.sum(t1 * C, axis=1, keepdims=True)

    hsE = jnp.dot(A, hs_ref[...], preferred_element_type=f32)
    pre = hsE + hdg_ref[...].astype(f32) + rflat * wr
    m1 = pre * _sig(pre)
    t = jnp.dot(m1.astype(jnp.bfloat16), we2t_ref[...],
                preferred_element_type=f32) + be2
    m2 = t * _sig(t)
    gate = _sig(jnp.sum(m2 * wa, axis=1, keepdims=True) + ba)
    m3 = m2 * gate
    agg = jnp.dot(S, m3, preferred_element_type=f32)

    hB = h_ref[...]
    t2 = (jnp.dot(hB, wn1ht_ref[...], preferred_element_type=f32)
          + jnp.dot(agg, wn1at_ref[...], preferred_element_type=f32) + bn1)
    o1 = t2 * _sig(t2)
    out_ref[...] = hB + jnp.dot(o1, wn2t_ref[...],
                                preferred_element_type=f32) + bn2


def _edge(hdg, hs, h, radial, we2t, wn1ht, wn1at, wn2t, vecS):
    grid = N_TOT // _NB
    fn = pl.pallas_call(
        _edge_body,
        grid=(grid,),
        in_specs=[
            pl.BlockSpec((_EB, D), lambda i: (i, 0)),
            pl.BlockSpec((_NB, D), lambda i: (i, 0)),
            pl.BlockSpec((_NB, D), lambda i: (i, 0)),
            pl.BlockSpec((_NB, 64), lambda i: (i, 0)),
            pl.BlockSpec((D, D), lambda i: (0, 0)),
            pl.BlockSpec((D, D), lambda i: (0, 0)),
            pl.BlockSpec((D, D), lambda i: (0, 0)),
            pl.BlockSpec((D, D), lambda i: (0, 0)),
            pl.BlockSpec((8, D), lambda i: (0, 0)),
        ],
        out_specs=pl.BlockSpec((_NB, D), lambda i: (i, 0)),
        out_shape=jax.ShapeDtypeStruct((N_TOT, D), jnp.float32),
    )
    return fn(hdg, hs, h, radial, we2t, wn1ht, wn1at, wn2t, vecS)


def _ln(x, g, b):
    mu = jnp.mean(x, axis=1, keepdims=True)
    var = jnp.mean((x - mu) ** 2, axis=1, keepdims=True)
    return (x - mu) / jnp.sqrt(var + 1e-5) * g + b


def _post_body(h_ref, x_ref, wqt_ref, wkt_ref, wvt_ref, wot_ref,
               wf1t_ref, wf2t_ref, bias_ref, bf1_ref, ln_ref, out_ref):
    f32 = jnp.float32
    nh, hd = 8, 32
    inv_s = 1.0 / (hd ** 0.5)

    h0 = h_ref[...] + x_ref[...]
    h1 = _ln(h0, ln_ref[0:1, :], ln_ref[1:2, :])

    q = jnp.dot(h1, wqt_ref[...], preferred_element_type=f32) + bias_ref[0:1, :]
    kk = jnp.dot(h1, wkt_ref[...], preferred_element_type=f32) + bias_ref[1:2, :]
    v = jnp.dot(h1, wvt_ref[...], preferred_element_type=f32) + bias_ref[2:3, :]

    def attend(qm, km, vm):
        outs = []
        for head in range(nh):
            s0 = head * hd
            qh = qm[:, s0:s0 + hd]
            kh = km[:, s0:s0 + hd]
            vh = vm[:, s0:s0 + hd]
            sc = lax.dot_general(qh, kh, (((1,), (1,)), ((), ())),
                                 preferred_element_type=f32) * inv_s
            sc = sc - jnp.max(sc, axis=1, keepdims=True)
            e = jnp.exp(sc)
            a = e / jnp.sum(e, axis=1, keepdims=True)
            outs.append(jnp.dot(a, vh, preferred_element_type=f32))
        return jnp.concatenate(outs, axis=1)

    att_rec = attend(q[:N_REC], kk[N_REC:], v[N_REC:])
    att_lig = attend(q[N_REC:], kk[:N_REC], v[:N_REC])
    att = jnp.concatenate([att_rec, att_lig], axis=0)
    att = jnp.dot(att, wot_ref[...], preferred_element_type=f32) + bias_ref[3:4, :]

    h2 = _ln(att + h1, ln_ref[2:3, :], ln_ref[3:4, :])
    f = jnp.dot(h2, wf1t_ref[...], preferred_element_type=f32) + bf1_ref[0:1, :]
    f = f * _sig(f)
    f2 = jnp.dot(f, wf2t_ref[...], preferred_element_type=f32) + bias_ref[4:5, :]
    out_ref[...] = _ln(f2 + h2, ln_ref[4:5, :], ln_ref[5:6, :])


def _post(h, xin, wqt, wkt, wvt, wot, wf1t, wf2t, biasP, bf1, lnS):
    fn = pl.pallas_call(
        _post_body,
        out_shape=jax.ShapeDtypeStruct((N_TOT, D), jnp.float32),
    )
    return fn(h, xin, wqt, wkt, wvt, wot, wf1t, wf2t, biasP, bf1, lnS)


def _gumbel_table(key, n):
    u = jax.random.uniform(key, (n, n - KNN), minval=1e-9, maxval=1.0)
    g = -jnp.log(-jnp.log(u))
    w = ((KNN + 1 + (n - KNN) + 127) // 128 + 1) * 128
    gp = jnp.zeros((n, w), jnp.float32)
    return gp.at[:, KNN + 1:KNN + 1 + (n - KNN)].set(g), w


def _pts_forms(pos):
    p = pos[:, 1]
    pts = jnp.pad(p, ((0, 0), (0, 5)))
    return pts, pts.T.reshape(8, -1)


def kernel(rec_x, lig_x, rec_pos, lig_pos, params):
    p = params
    k1, k2 = jax.random.split(jax.random.key(42))
    gp_rec, w_rec = _gumbel_table(k1, N_REC)
    gp_lig, w_lig = _gumbel_table(k2, N_LIG)

    pts_r, ptsT_r = _pts_forms(rec_pos)
    pts_l, ptsT_l = _pts_forms(lig_pos)

    idx_r, rad_r = _select(pts_r, ptsT_r, gp_rec, N_REC, w_rec)
    idx_l, rad_l = _select(pts_l, ptsT_l, gp_lig, N_LIG, w_lig)

    idx_all = jnp.concatenate([
        idx_r[:, :DEG].reshape(-1),
        idx_l[:, :DEG].reshape(-1) + N_REC,
    ]).astype(jnp.int32)
    radial = jnp.concatenate([rad_r, rad_l], axis=0)

    h = jnp.concatenate([rec_x, lig_x], axis=0)
    for lp in p["egnn"]:
        wst = lp["We1"][:, :D].T
        wdt = lp["We1"][:, D:2 * D].T
        wr = lp["We1"][:, 2 * D]
        projB = jnp.zeros((8, D), jnp.float32).at[0].set(lp["be1"])
        vecS = (jnp.zeros((8, D), jnp.float32)
                .at[0].set(lp["be2"])
                .at[1].set(lp["bn1"])
                .at[2].set(lp["bn2"])
                .at[3].set(lp["ba"][0])
                .at[4].set(wr)
                .at[5].set(lp["Wa"][0]))
        hs, hd = _proj(h, wst, wdt, projB)
        hd_packed = lax.bitcast_convert_type(
            hd.reshape(N_TOT, D // 2, 2), jnp.float32)
        hdg_words = _sc_gather(hd_packed, idx_all)
        hdg = lax.bitcast_convert_type(hdg_words, jnp.bfloat16
                                       ).reshape(E_TOT, D)
        h = _edge(hdg, hs, h, radial, lp["We2"].T.astype(jnp.bfloat16),
                  lp["Wn1"][:, :D].T, lp["Wn1"][:, D:].T, lp["Wn2"].T, vecS)

    xin = jnp.concatenate([rec_x, lig_x], axis=0)
    biasP = (jnp.zeros((8, D), jnp.float32)
             .at[0].set(p["bin"][:D])
             .at[1].set(p["bin"][D:2 * D])
             .at[2].set(p["bin"][2 * D:])
             .at[3].set(p["bout"])
             .at[4].set(p["bf2"]))
    bf1 = jnp.zeros((8, 2 * D), jnp.float32).at[0].set(p["bf1"])
    lnS = (jnp.zeros((8, D), jnp.float32)
           .at[0].set(p["n1g"]).at[1].set(p["n1b"])
           .at[2].set(p["n2g"]).at[3].set(p["n2b"])
           .at[4].set(p["n3g"]).at[5].set(p["n3b"]))
    out = _post(h, xin,
                p["Win"][:D].T, p["Win"][D:2 * D].T, p["Win"][2 * D:].T,
                p["Wout"].T, p["Wf1"].T, p["Wf2"].T, biasP, bf1, lnS)
    return out


# speedup vs baseline: 2.2906x; 2.2906x over previous
import functools

import jax
import jax.numpy as jnp
from jax import lax
from jax.experimental import pallas as pl
from jax.experimental.pallas import tpu as pltpu
from jax.experimental.pallas import tpu_sc as plsc

D = 256
KNN = 20
SAMPLE = 40
DEG = KNN + SAMPLE
N_REC = 1024
N_LIG = 256
N_TOT = N_REC + N_LIG
E_TOT = N_TOT * DEG

_NEG_INF = float("-inf")
_POS_INF = float("inf")


def _sig(x):
    return 1.0 / (1.0 + jnp.exp(-x))


def _select_body(N, W, B, pts_ref, ptsT_ref, gp_ref, idx_ref, rad_ref):
    px = pts_ref[:, 0:1]
    py = pts_ref[:, 1:2]
    pz = pts_ref[:, 2:3]
    qx = ptsT_ref[0:1, :]
    qy = ptsT_ref[1:2, :]
    qz = ptsT_ref[2:3, :]
    dx = px - qx
    dy = py - qy
    dz = pz - qz
    d2 = (dx * dx + dy * dy) + dz * dz
    dist = jnp.sqrt(d2 + 1e-12)
    iota = lax.broadcasted_iota(jnp.int32, (B, N), 1)

    work = dist
    isknn = jnp.zeros((B, N), jnp.bool_)
    cnt = jnp.zeros((B, N), jnp.float32)
    for k in range(KNN):
        mv = jnp.min(work, axis=1, keepdims=True)
        idxk = jnp.min(jnp.where(work == mv, iota, N), axis=1, keepdims=True)
        sel = iota == idxk
        rad_ref[:, k:k + 1] = mv * mv - 1e-12
        idx_ref[:, k:k + 1] = idxk
        work = jnp.where(sel, _POS_INF, work)
        isknn = jnp.logical_or(isknn, sel)
        cnt = cnt + (iota >= idxk).astype(jnp.float32)

    base = jnp.log(1.0 / (dist * dist * dist))
    gexp = jnp.zeros((B, N), jnp.float32)
    for cc in range(KNN + 1):
        sl = gp_ref[:, KNN + 1 - cc: KNN + 1 - cc + N]
        gexp = jnp.where(cnt == cc, sl, gexp)
    score = jnp.where(isknn, _NEG_INF, base + gexp)

    for k in range(SAMPLE):
        mv = jnp.max(score, axis=1, keepdims=True)
        idxk = jnp.min(jnp.where(score == mv, iota, N), axis=1, keepdims=True)
        sel = iota == idxk
        rad_ref[:, KNN + k:KNN + k + 1] = jnp.sum(
            jnp.where(sel, d2, 0.0), axis=1, keepdims=True)
        idx_ref[:, KNN + k:KNN + k + 1] = idxk
        score = jnp.where(sel, _NEG_INF, score)

    idx_ref[:, DEG:] = jnp.zeros((B, 64 - DEG), jnp.int32)
    rad_ref[:, DEG:] = jnp.zeros((B, 64 - DEG), jnp.float32)


def _select(pts, ptsT, gp, N, W, B=128):
    grid = N // B
    fn = pl.pallas_call(
        functools.partial(_select_body, N, W, B),
        grid=(grid,),
        in_specs=[
            pl.BlockSpec((B, 8), lambda i: (i, 0)),
            pl.BlockSpec((8, N), lambda i: (0, 0)),
            pl.BlockSpec((B, W), lambda i: (i, 0)),
        ],
        out_specs=[
            pl.BlockSpec((B, 64), lambda i: (i, 0)),
            pl.BlockSpec((B, 64), lambda i: (i, 0)),
        ],
        out_shape=[
            jax.ShapeDtypeStruct((N, 64), jnp.int32),
            jax.ShapeDtypeStruct((N, 64), jnp.float32),
        ],
    )
    return fn(pts, ptsT, gp)


def _proj_body(h_ref, wst_ref, wdt_ref, bias_ref, hs_ref, hd_ref):
    hh = h_ref[...]
    hs_ref[...] = jnp.dot(hh, wst_ref[...],
                          preferred_element_type=jnp.float32) + bias_ref[0:1, :]
    hd = jnp.dot(hh, wdt_ref[...], preferred_element_type=jnp.float32)
    lo = lax.bitcast_convert_type(hd[:, :D // 2].astype(jnp.bfloat16),
                                  jnp.uint16).astype(jnp.uint32)
    hi = lax.bitcast_convert_type(hd[:, D // 2:].astype(jnp.bfloat16),
                                  jnp.uint16).astype(jnp.uint32)
    hd_ref[...] = lax.bitcast_convert_type((hi << 16) | lo, jnp.float32)


def _proj(h, wst, wdt, biasS):
    fn = pl.pallas_call(
        _proj_body,
        out_shape=[
            jax.ShapeDtypeStruct((N_TOT, D), jnp.float32),
            jax.ShapeDtypeStruct((N_TOT, D // 2), jnp.float32),
        ],
    )
    return fn(h, wst, wdt, biasS)


_SC_WORKERS = 32
_SC_CHUNK = 120
_SC_PER_W = E_TOT // _SC_WORKERS
_SC_NCHUNK = _SC_PER_W // _SC_CHUNK


def _sc_gather_body(table_hbm, idx_hbm, out_hbm, idx_v, rows_v, sem):
    wid = lax.axis_index("s") * 2 + lax.axis_index("c")
    base = wid * _SC_PER_W

    def body(c, carry):
        off = base + c * _SC_CHUNK
        pltpu.sync_copy(idx_hbm.at[pl.ds(off, _SC_CHUNK)], idx_v)
        pltpu.async_copy(table_hbm.at[idx_v], rows_v, sem).wait()
        pltpu.sync_copy(rows_v, out_hbm.at[pl.ds(off, _SC_CHUNK)])
        return carry

    lax.fori_loop(0, _SC_NCHUNK, body, 0)


def _sc_gather(table, idx):
    mesh = plsc.VectorSubcoreMesh(core_axis_name="c", subcore_axis_name="s")
    fn = pl.kernel(
        _sc_gather_body,
        out_type=jax.ShapeDtypeStruct((E_TOT, D // 2), jnp.float32),
        mesh=mesh,
        scratch_types=[
            pltpu.VMEM((_SC_CHUNK,), jnp.int32),
            pltpu.VMEM((_SC_CHUNK, D // 2), jnp.float32),
            pltpu.SemaphoreType.DMA,
        ],
    )
    return fn(table, idx)


_NB = 16
_EB = _NB * DEG


def _edge_body(hdg_ref, hs_ref, h_ref, rad_ref, we2t_ref,
               wn1ht_ref, wn1at_ref, wn2t_ref, vec_ref, out_ref):
    f32 = jnp.float32
    row_node = lax.broadcasted_iota(jnp.int32, (_EB, _NB), 0) // DEG
    col16 = lax.broadcasted_iota(jnp.int32, (_EB, _NB), 1)
    A = (row_node == col16).astype(f32)
    rowm = lax.broadcasted_iota(jnp.int32, (_EB, DEG), 0) % DEG
    colm = lax.broadcasted_iota(jnp.int32, (_EB, DEG), 1)
    C = (rowm == colm).astype(f32)
    srow = lax.broadcasted_iota(jnp.int32, (_NB, _EB), 0)
    scol = lax.broadcasted_iota(jnp.int32, (_NB, _EB), 1) // DEG
    S = (srow == scol).astype(f32)

    be2 = vec_ref[0:1, :]
    bn1 = vec_ref[1:2, :]
    bn2 = vec_ref[2:3, :]
    ba = vec_ref[3:4, 0:1]
    wr = vec_ref[4:5, :]
    wa = vec_ref[5:6, :]

    radB = rad_ref[:, :DEG]
    t1 = jnp.dot(A, radB, preferred_element_type=f32)
    rflat = jnp.sum(t1 * C, axis=1, keepdims=True)

    w = lax.bitcast_convert_type(hdg_ref[...], jnp.uint32)
    lo = lax.bitcast_convert_type((w & 0xFFFF).astype(jnp.uint16),
                                  jnp.bfloat16)
    hi = lax.bitcast_convert_type((w >> 16).astype(jnp.uint16), jnp.bfloat16)
    hdg = jnp.concatenate([lo, hi], axis=1)
    hsE = jnp.dot(A, hs_ref[...], preferred_element_type=f32)
    pre = hsE + hdg.astype(f32) + rflat * wr
    m1 = pre * _sig(pre)
    t = jnp.dot(m1.astype(jnp.bfloat16), we2t_ref[...],
                preferred_element_type=f32) + be2
    m2 = t * _sig(t)
    gate = _sig(jnp.sum(m2 * wa, axis=1, keepdims=True) + ba)
    m3 = m2 * gate
    agg = jnp.dot(S, m3, preferred_element_type=f32)

    hB = h_ref[...]
    t2 = (jnp.dot(hB, wn1ht_ref[...], preferred_element_type=f32)
          + jnp.dot(agg, wn1at_ref[...], preferred_element_type=f32) + bn1)
    o1 = t2 * _sig(t2)
    out_ref[...] = hB + jnp.dot(o1, wn2t_ref[...],
                                preferred_element_type=f32) + bn2


def _edge(hdg, hs, h, radial, we2t, wn1ht, wn1at, wn2t, vecS):
    grid = N_TOT // _NB
    fn = pl.pallas_call(
        _edge_body,
        grid=(grid,),
        in_specs=[
            pl.BlockSpec((_EB, D // 2), lambda i: (i, 0)),
            pl.BlockSpec((_NB, D), lambda i: (i, 0)),
            pl.BlockSpec((_NB, D), lambda i: (i, 0)),
            pl.BlockSpec((_NB, 64), lambda i: (i, 0)),
            pl.BlockSpec((D, D), lambda i: (0, 0)),
            pl.BlockSpec((D, D), lambda i: (0, 0)),
            pl.BlockSpec((D, D), lambda i: (0, 0)),
            pl.BlockSpec((D, D), lambda i: (0, 0)),
            pl.BlockSpec((8, D), lambda i: (0, 0)),
        ],
        out_specs=pl.BlockSpec((_NB, D), lambda i: (i, 0)),
        out_shape=jax.ShapeDtypeStruct((N_TOT, D), jnp.float32),
    )
    return fn(hdg, hs, h, radial, we2t, wn1ht, wn1at, wn2t, vecS)


def _ln(x, g, b):
    mu = jnp.mean(x, axis=1, keepdims=True)
    var = jnp.mean((x - mu) ** 2, axis=1, keepdims=True)
    return (x - mu) / jnp.sqrt(var + 1e-5) * g + b


def _post_body(h_ref, x_ref, wqt_ref, wkt_ref, wvt_ref, wot_ref,
               wf1t_ref, wf2t_ref, bias_ref, bf1_ref, ln_ref, out_ref):
    f32 = jnp.float32
    nh, hd = 8, 32
    inv_s = 1.0 / (hd ** 0.5)

    h0 = h_ref[...] + x_ref[...]
    h1 = _ln(h0, ln_ref[0:1, :], ln_ref[1:2, :])

    q = jnp.dot(h1, wqt_ref[...], preferred_element_type=f32) + bias_ref[0:1, :]
    kk = jnp.dot(h1, wkt_ref[...], preferred_element_type=f32) + bias_ref[1:2, :]
    v = jnp.dot(h1, wvt_ref[...], preferred_element_type=f32) + bias_ref[2:3, :]

    def attend(qm, km, vm):
        outs = []
        for head in range(nh):
            s0 = head * hd
            qh = qm[:, s0:s0 + hd]
            kh = km[:, s0:s0 + hd]
            vh = vm[:, s0:s0 + hd]
            sc = lax.dot_general(qh, kh, (((1,), (1,)), ((), ())),
                                 preferred_element_type=f32) * inv_s
            sc = sc - jnp.max(sc, axis=1, keepdims=True)
            e = jnp.exp(sc)
            a = e / jnp.sum(e, axis=1, keepdims=True)
            outs.append(jnp.dot(a, vh, preferred_element_type=f32))
        return jnp.concatenate(outs, axis=1)

    att_rec = attend(q[:N_REC], kk[N_REC:], v[N_REC:])
    att_lig = attend(q[N_REC:], kk[:N_REC], v[:N_REC])
    att = jnp.concatenate([att_rec, att_lig], axis=0)
    att = jnp.dot(att, wot_ref[...], preferred_element_type=f32) + bias_ref[3:4, :]

    h2 = _ln(att + h1, ln_ref[2:3, :], ln_ref[3:4, :])
    f = jnp.dot(h2, wf1t_ref[...], preferred_element_type=f32) + bf1_ref[0:1, :]
    f = f * _sig(f)
    f2 = jnp.dot(f, wf2t_ref[...], preferred_element_type=f32) + bias_ref[4:5, :]
    out_ref[...] = _ln(f2 + h2, ln_ref[4:5, :], ln_ref[5:6, :])


def _post(h, xin, wqt, wkt, wvt, wot, wf1t, wf2t, biasP, bf1, lnS):
    fn = pl.pallas_call(
        _post_body,
        out_shape=jax.ShapeDtypeStruct((N_TOT, D), jnp.float32),
    )
    return fn(h, xin, wqt, wkt, wvt, wot, wf1t, wf2t, biasP, bf1, lnS)


def _gumbel_table(key, n):
    u = jax.random.uniform(key, (n, n - KNN), minval=1e-9, maxval=1.0)
    g = -jnp.log(-jnp.log(u))
    w = ((KNN + 1 + (n - KNN) + 127) // 128 + 1) * 128
    gp = jnp.zeros((n, w), jnp.float32)
    return gp.at[:, KNN + 1:KNN + 1 + (n - KNN)].set(g), w


def _pts_forms(pos):
    p = pos[:, 1]
    pts = jnp.pad(p, ((0, 0), (0, 5)))
    return pts, pts.T.reshape(8, -1)


def kernel(rec_x, lig_x, rec_pos, lig_pos, params):
    p = params
    k1, k2 = jax.random.split(jax.random.key(42))
    gp_rec, w_rec = _gumbel_table(k1, N_REC)
    gp_lig, w_lig = _gumbel_table(k2, N_LIG)

    pts_r, ptsT_r = _pts_forms(rec_pos)
    pts_l, ptsT_l = _pts_forms(lig_pos)

    idx_r, rad_r = _select(pts_r, ptsT_r, gp_rec, N_REC, w_rec)
    idx_l, rad_l = _select(pts_l, ptsT_l, gp_lig, N_LIG, w_lig)

    idx_all = jnp.concatenate([
        idx_r[:, :DEG].reshape(-1),
        idx_l[:, :DEG].reshape(-1) + N_REC,
    ]).astype(jnp.int32)
    radial = jnp.concatenate([rad_r, rad_l], axis=0)

    h = jnp.concatenate([rec_x, lig_x], axis=0)
    for lp in p["egnn"]:
        wst = lp["We1"][:, :D].T
        wdt = lp["We1"][:, D:2 * D].T
        wr = lp["We1"][:, 2 * D]
        projB = jnp.zeros((8, D), jnp.float32).at[0].set(lp["be1"])
        vecS = (jnp.zeros((8, D), jnp.float32)
                .at[0].set(lp["be2"])
                .at[1].set(lp["bn1"])
                .at[2].set(lp["bn2"])
                .at[3].set(lp["ba"][0])
                .at[4].set(wr)
                .at[5].set(lp["Wa"][0]))
        hs, hd_packed = _proj(h, wst, wdt, projB)
        hdg = _sc_gather(hd_packed, idx_all)
        h = _edge(hdg, hs, h, radial, lp["We2"].T.astype(jnp.bfloat16),
                  lp["Wn1"][:, :D].T, lp["Wn1"][:, D:].T, lp["Wn2"].T, vecS)

    xin = jnp.concatenate([rec_x, lig_x], axis=0)
    biasP = (jnp.zeros((8, D), jnp.float32)
             .at[0].set(p["bin"][:D])
             .at[1].set(p["bin"][D:2 * D])
             .at[2].set(p["bin"][2 * D:])
             .at[3].set(p["bout"])
             .at[4].set(p["bf2"]))
    bf1 = jnp.zeros((8, 2 * D), jnp.float32).at[0].set(p["bf1"])
    lnS = (jnp.zeros((8, D), jnp.float32)
           .at[0].set(p["n1g"]).at[1].set(p["n1b"])
           .at[2].set(p["n2g"]).at[3].set(p["n2b"])
           .at[4].set(p["n3g"]).at[5].set(p["n3b"]))
    out = _post(h, xin,
                p["Win"][:D].T, p["Win"][D:2 * D].T, p["Win"][2 * D:].T,
                p["Wout"].T, p["Wf1"].T, p["Wf2"].T, biasP, bf1, lnS)
    return out


# speedup vs baseline: 2.5013x; 1.0920x over previous
import functools

import jax
import jax.numpy as jnp
from jax import lax
from jax.experimental import pallas as pl
from jax.experimental.pallas import tpu as pltpu
from jax.experimental.pallas import tpu_sc as plsc

D = 256
KNN = 20
SAMPLE = 40
DEG = KNN + SAMPLE
N_REC = 1024
N_LIG = 256
N_TOT = N_REC + N_LIG
E_TOT = N_TOT * DEG

_NEG_INF = float("-inf")
_POS_INF = float("inf")


def _sig(x):
    return 1.0 / (1.0 + jnp.exp(-x))


def _select_body(N, W, B, pts_ref, ptsT_ref, gp_ref, idx_ref, rad_ref):
    px = pts_ref[:, 0:1]
    py = pts_ref[:, 1:2]
    pz = pts_ref[:, 2:3]
    qx = ptsT_ref[0:1, :]
    qy = ptsT_ref[1:2, :]
    qz = ptsT_ref[2:3, :]
    dx = px - qx
    dy = py - qy
    dz = pz - qz
    d2 = (dx * dx + dy * dy) + dz * dz
    dist = jnp.sqrt(d2 + 1e-12)
    iota = lax.broadcasted_iota(jnp.int32, (B, N), 1)

    work = dist
    cnt = jnp.zeros((B, N), jnp.float32)
    for k in range(KNN):
        mv = jnp.min(work, axis=1, keepdims=True)
        idxk = jnp.min(jnp.where(work == mv, iota, N), axis=1, keepdims=True)
        rad_ref[:, k:k + 1] = mv * mv - 1e-12
        idx_ref[:, k:k + 1] = idxk
        work = jnp.where(iota == idxk, _POS_INF, work)
        cnt = cnt + (iota >= idxk).astype(jnp.float32)
    isknn = work == _POS_INF

    base = jnp.log(1.0 / (dist * dist * dist))
    gexp = jnp.zeros((B, N), jnp.float32)
    for cc in range(KNN + 1):
        sl = gp_ref[:, KNN + 1 - cc: KNN + 1 - cc + N]
        gexp = jnp.where(cnt == cc, sl, gexp)
    score = jnp.where(isknn, _NEG_INF, base + gexp)

    for k in range(SAMPLE):
        mv = jnp.max(score, axis=1, keepdims=True)
        idxk = jnp.min(jnp.where(score == mv, iota, N), axis=1, keepdims=True)
        sel = iota == idxk
        rad_ref[:, KNN + k:KNN + k + 1] = jnp.sum(
            jnp.where(sel, d2, 0.0), axis=1, keepdims=True)
        idx_ref[:, KNN + k:KNN + k + 1] = idxk
        score = jnp.where(sel, _NEG_INF, score)

    idx_ref[:, DEG:] = jnp.zeros((B, 64 - DEG), jnp.int32)
    rad_ref[:, DEG:] = jnp.zeros((B, 64 - DEG), jnp.float32)


def _select(pts, ptsT, gp, N, W, B=128):
    grid = N // B
    fn = pl.pallas_call(
        functools.partial(_select_body, N, W, B),
        grid=(grid,),
        in_specs=[
            pl.BlockSpec((B, 8), lambda i: (i, 0)),
            pl.BlockSpec((8, N), lambda i: (0, 0)),
            pl.BlockSpec((B, W), lambda i: (i, 0)),
        ],
        out_specs=[
            pl.BlockSpec((B, 64), lambda i: (i, 0)),
            pl.BlockSpec((B, 64), lambda i: (i, 0)),
        ],
        out_shape=[
            jax.ShapeDtypeStruct((N, 64), jnp.int32),
            jax.ShapeDtypeStruct((N, 64), jnp.float32),
        ],
    )
    return fn(pts, ptsT, gp)


def _proj_body(h_ref, wst_ref, wdt_ref, bias_ref, hs_ref, hd_ref):
    hh = h_ref[...]
    hs_ref[...] = jnp.dot(hh, wst_ref[...],
                          preferred_element_type=jnp.float32) + bias_ref[0:1, :]
    hd = jnp.dot(hh, wdt_ref[...], preferred_element_type=jnp.float32)
    lo = lax.bitcast_convert_type(hd[:, :D // 2].astype(jnp.bfloat16),
                                  jnp.uint16).astype(jnp.uint32)
    hi = lax.bitcast_convert_type(hd[:, D // 2:].astype(jnp.bfloat16),
                                  jnp.uint16).astype(jnp.uint32)
    hd_ref[...] = lax.bitcast_convert_type((hi << 16) | lo, jnp.float32)


def _proj(h, wst, wdt, biasS):
    fn = pl.pallas_call(
        _proj_body,
        out_shape=[
            jax.ShapeDtypeStruct((N_TOT, D), jnp.float32),
            jax.ShapeDtypeStruct((N_TOT, D // 2), jnp.float32),
        ],
    )
    return fn(h, wst, wdt, biasS)


_SC_WORKERS = 32
_SC_CHUNK = 120
_SC_PER_W = E_TOT // _SC_WORKERS
_SC_NCHUNK = _SC_PER_W // _SC_CHUNK


def _sc_gather_body(table_hbm, idx_hbm, out_hbm, idx_v, rows_v, sem):
    wid = lax.axis_index("s") * 2 + lax.axis_index("c")
    base = wid * _SC_PER_W

    def body(c, carry):
        off = base + c * _SC_CHUNK
        pltpu.sync_copy(idx_hbm.at[pl.ds(off, _SC_CHUNK)], idx_v)
        pltpu.async_copy(table_hbm.at[idx_v], rows_v, sem).wait()
        pltpu.sync_copy(rows_v, out_hbm.at[pl.ds(off, _SC_CHUNK)])
        return carry

    lax.fori_loop(0, _SC_NCHUNK, body, 0)


def _sc_gather(table, idx):
    mesh = plsc.VectorSubcoreMesh(core_axis_name="c", subcore_axis_name="s")
    fn = pl.kernel(
        _sc_gather_body,
        out_type=jax.ShapeDtypeStruct((E_TOT, D // 2), jnp.float32),
        mesh=mesh,
        scratch_types=[
            pltpu.VMEM((_SC_CHUNK,), jnp.int32),
            pltpu.VMEM((_SC_CHUNK, D // 2), jnp.float32),
            pltpu.SemaphoreType.DMA,
        ],
    )
    return fn(table, idx)


_NB = 32
_EB = _NB * DEG


def _edge_body(hdg_ref, hs_ref, h_ref, rad_ref, we2t_ref,
               wn1ht_ref, wn1at_ref, wn2t_ref, wstn_ref, wdtn_ref, vec_ref,
               out_ref, hsn_ref, hdn_ref):
    f32 = jnp.float32
    row_node = lax.broadcasted_iota(jnp.int32, (_EB, _NB), 0) // DEG
    col16 = lax.broadcasted_iota(jnp.int32, (_EB, _NB), 1)
    A = (row_node == col16).astype(f32)
    rowm = lax.broadcasted_iota(jnp.int32, (_EB, DEG), 0) % DEG
    colm = lax.broadcasted_iota(jnp.int32, (_EB, DEG), 1)
    C = (rowm == colm).astype(f32)
    srow = lax.broadcasted_iota(jnp.int32, (_NB, _EB), 0)
    scol = lax.broadcasted_iota(jnp.int32, (_NB, _EB), 1) // DEG
    S = (srow == scol).astype(f32)

    be2 = vec_ref[0:1, :]
    bn1 = vec_ref[1:2, :]
    bn2 = vec_ref[2:3, :]
    ba = vec_ref[3:4, 0:1]
    wr = vec_ref[4:5, :]
    wa = vec_ref[5:6, :]

    radB = rad_ref[:, :DEG]
    t1 = jnp.dot(A, radB, preferred_element_type=f32)
    rflat = jnp.sum(t1 * C, axis=1, keepdims=True)

    w = lax.bitcast_convert_type(hdg_ref[...], jnp.uint32)
    lo = lax.bitcast_convert_type((w & 0xFFFF).astype(jnp.uint16),
                                  jnp.bfloat16)
    hi = lax.bitcast_convert_type((w >> 16).astype(jnp.uint16), jnp.bfloat16)
    hdg = jnp.concatenate([lo, hi], axis=1)
    hsE = jnp.dot(A, hs_ref[...], preferred_element_type=f32)
    pre = hsE + hdg.astype(f32) + rflat * wr
    m1 = pre * _sig(pre)
    t = jnp.dot(m1.astype(jnp.bfloat16), we2t_ref[...],
                preferred_element_type=f32) + be2
    m2 = t * _sig(t)
    gate = _sig(jnp.sum(m2 * wa, axis=1, keepdims=True) + ba)
    m3 = m2 * gate
    agg = jnp.dot(S, m3, preferred_element_type=f32)

    hB = h_ref[...]
    t2 = (jnp.dot(hB, wn1ht_ref[...], preferred_element_type=f32)
          + jnp.dot(agg, wn1at_ref[...], preferred_element_type=f32) + bn1)
    o1 = t2 * _sig(t2)
    hnew = hB + jnp.dot(o1, wn2t_ref[...], preferred_element_type=f32) + bn2
    out_ref[...] = hnew

    be1n = vec_ref[6:7, :]
    hsn_ref[...] = jnp.dot(hnew, wstn_ref[...],
                           preferred_element_type=f32) + be1n
    hdn = jnp.dot(hnew, wdtn_ref[...], preferred_element_type=f32)
    lo2 = lax.bitcast_convert_type(hdn[:, :D // 2].astype(jnp.bfloat16),
                                   jnp.uint16).astype(jnp.uint32)
    hi2 = lax.bitcast_convert_type(hdn[:, D // 2:].astype(jnp.bfloat16),
                                   jnp.uint16).astype(jnp.uint32)
    hdn_ref[...] = lax.bitcast_convert_type((hi2 << 16) | lo2, jnp.float32)


def _edge(hdg, hs, h, radial, we2t, wn1ht, wn1at, wn2t, wstn, wdtn, vecS):
    grid = N_TOT // _NB
    fn = pl.pallas_call(
        _edge_body,
        grid=(grid,),
        in_specs=[
            pl.BlockSpec((_EB, D // 2), lambda i: (i, 0)),
            pl.BlockSpec((_NB, D), lambda i: (i, 0)),
            pl.BlockSpec((_NB, D), lambda i: (i, 0)),
            pl.BlockSpec((_NB, 64), lambda i: (i, 0)),
            pl.BlockSpec((D, D), lambda i: (0, 0)),
            pl.BlockSpec((D, D), lambda i: (0, 0)),
            pl.BlockSpec((D, D), lambda i: (0, 0)),
            pl.BlockSpec((D, D), lambda i: (0, 0)),
            pl.BlockSpec((D, D), lambda i: (0, 0)),
            pl.BlockSpec((D, D), lambda i: (0, 0)),
            pl.BlockSpec((8, D), lambda i: (0, 0)),
        ],
        out_specs=[
            pl.BlockSpec((_NB, D), lambda i: (i, 0)),
            pl.BlockSpec((_NB, D), lambda i: (i, 0)),
            pl.BlockSpec((_NB, D // 2), lambda i: (i, 0)),
        ],
        out_shape=[
            jax.ShapeDtypeStruct((N_TOT, D), jnp.float32),
            jax.ShapeDtypeStruct((N_TOT, D), jnp.float32),
            jax.ShapeDtypeStruct((N_TOT, D // 2), jnp.float32),
        ],
    )
    return fn(hdg, hs, h, radial, we2t, wn1ht, wn1at, wn2t, wstn, wdtn, vecS)


def _ln(x, g, b):
    mu = jnp.mean(x, axis=1, keepdims=True)
    var = jnp.mean((x - mu) ** 2, axis=1, keepdims=True)
    return (x - mu) / jnp.sqrt(var + 1e-5) * g + b


def _post_body(h_ref, x_ref, wqt_ref, wkt_ref, wvt_ref, wot_ref,
               wf1t_ref, wf2t_ref, bias_ref, bf1_ref, ln_ref, out_ref):
    f32 = jnp.float32
    nh, hd = 8, 32
    inv_s = 1.0 / (hd ** 0.5)

    h0 = h_ref[...] + x_ref[...]
    h1 = _ln(h0, ln_ref[0:1, :], ln_ref[1:2, :])

    q = jnp.dot(h1, wqt_ref[...], preferred_element_type=f32) + bias_ref[0:1, :]
    kk = jnp.dot(h1, wkt_ref[...], preferred_element_type=f32) + bias_ref[1:2, :]
    v = jnp.dot(h1, wvt_ref[...], preferred_element_type=f32) + bias_ref[2:3, :]

    def attend(qm, km, vm):
        outs = []
        for head in range(nh):
            s0 = head * hd
            qh = qm[:, s0:s0 + hd]
            kh = km[:, s0:s0 + hd]
            vh = vm[:, s0:s0 + hd]
            sc = lax.dot_general(qh, kh, (((1,), (1,)), ((), ())),
                                 preferred_element_type=f32) * inv_s
            sc = sc - jnp.max(sc, axis=1, keepdims=True)
            e = jnp.exp(sc)
            a = e / jnp.sum(e, axis=1, keepdims=True)
            outs.append(jnp.dot(a, vh, preferred_element_type=f32))
        return jnp.concatenate(outs, axis=1)

    att_rec = attend(q[:N_REC], kk[N_REC:], v[N_REC:])
    att_lig = attend(q[N_REC:], kk[:N_REC], v[:N_REC])
    att = jnp.concatenate([att_rec, att_lig], axis=0)
    att = jnp.dot(att, wot_ref[...], preferred_element_type=f32) + bias_ref[3:4, :]

    h2 = _ln(att + h1, ln_ref[2:3, :], ln_ref[3:4, :])
    f = jnp.dot(h2, wf1t_ref[...], preferred_element_type=f32) + bf1_ref[0:1, :]
    f = f * _sig(f)
    f2 = jnp.dot(f, wf2t_ref[...], preferred_element_type=f32) + bias_ref[4:5, :]
    out_ref[...] = _ln(f2 + h2, ln_ref[4:5, :], ln_ref[5:6, :])


def _post(h, xin, wqt, wkt, wvt, wot, wf1t, wf2t, biasP, bf1, lnS):
    fn = pl.pallas_call(
        _post_body,
        out_shape=jax.ShapeDtypeStruct((N_TOT, D), jnp.float32),
    )
    return fn(h, xin, wqt, wkt, wvt, wot, wf1t, wf2t, biasP, bf1, lnS)


def _gumbel_table(key, n):
    u = jax.random.uniform(key, (n, n - KNN), minval=1e-9, maxval=1.0)
    g = -jnp.log(-jnp.log(u))
    w = ((KNN + 1 + (n - KNN) + 127) // 128 + 1) * 128
    gp = jnp.zeros((n, w), jnp.float32)
    return gp.at[:, KNN + 1:KNN + 1 + (n - KNN)].set(g), w


def _pts_forms(pos):
    p = pos[:, 1]
    pts = jnp.pad(p, ((0, 0), (0, 5)))
    return pts, pts.T.reshape(8, -1)


def kernel(rec_x, lig_x, rec_pos, lig_pos, params):
    p = params
    k1, k2 = jax.random.split(jax.random.key(42))
    gp_rec, w_rec = _gumbel_table(k1, N_REC)
    gp_lig, w_lig = _gumbel_table(k2, N_LIG)

    pts_r, ptsT_r = _pts_forms(rec_pos)
    pts_l, ptsT_l = _pts_forms(lig_pos)

    idx_r, rad_r = _select(pts_r, ptsT_r, gp_rec, N_REC, w_rec)
    idx_l, rad_l = _select(pts_l, ptsT_l, gp_lig, N_LIG, w_lig)

    idx_all = jnp.concatenate([
        idx_r[:, :DEG].reshape(-1),
        idx_l[:, :DEG].reshape(-1) + N_REC,
    ]).astype(jnp.int32)
    radial = jnp.concatenate([rad_r, rad_l], axis=0)

    h = jnp.concatenate([rec_x, lig_x], axis=0)
    layers = p["egnn"]
    wst0 = layers[0]["We1"][:, :D].T
    wdt0 = layers[0]["We1"][:, D:2 * D].T
    projB = jnp.zeros((8, D), jnp.float32).at[0].set(layers[0]["be1"])
    hs, hd_packed = _proj(h, wst0, wdt0, projB)
    for li, lp in enumerate(layers):
        nxt = layers[li + 1] if li + 1 < len(layers) else layers[0]
        vecS = (jnp.zeros((8, D), jnp.float32)
                .at[0].set(lp["be2"])
                .at[1].set(lp["bn1"])
                .at[2].set(lp["bn2"])
                .at[3].set(lp["ba"][0])
                .at[4].set(lp["We1"][:, 2 * D])
                .at[5].set(lp["Wa"][0])
                .at[6].set(nxt["be1"]))
        hdg = _sc_gather(hd_packed, idx_all)
        h, hs, hd_packed = _edge(
            hdg, hs, h, radial, lp["We2"].T.astype(jnp.bfloat16),
            lp["Wn1"][:, :D].T, lp["Wn1"][:, D:].T, lp["Wn2"].T,
            nxt["We1"][:, :D].T, nxt["We1"][:, D:2 * D].T, vecS)

    xin = jnp.concatenate([rec_x, lig_x], axis=0)
    biasP = (jnp.zeros((8, D), jnp.float32)
             .at[0].set(p["bin"][:D])
             .at[1].set(p["bin"][D:2 * D])
             .at[2].set(p["bin"][2 * D:])
             .at[3].set(p["bout"])
             .at[4].set(p["bf2"]))
    bf1 = jnp.zeros((8, 2 * D), jnp.float32).at[0].set(p["bf1"])
    lnS = (jnp.zeros((8, D), jnp.float32)
           .at[0].set(p["n1g"]).at[1].set(p["n1b"])
           .at[2].set(p["n2g"]).at[3].set(p["n2b"])
           .at[4].set(p["n3g"]).at[5].set(p["n3b"]))
    out = _post(h, xin,
                p["Win"][:D].T, p["Win"][D:2 * D].T, p["Win"][2 * D:].T,
                p["Wout"].T, p["Wf1"].T, p["Wf2"].T, biasP, bf1, lnS)
    return out


# speedup vs baseline: 2.6631x; 1.0647x over previous
import functools

import jax
import jax.numpy as jnp
from jax import lax
from jax.experimental import pallas as pl
from jax.experimental.pallas import tpu as pltpu
from jax.experimental.pallas import tpu_sc as plsc

D = 256
KNN = 20
SAMPLE = 40
DEG = KNN + SAMPLE
N_REC = 1024
N_LIG = 256
N_TOT = N_REC + N_LIG
E_TOT = N_TOT * DEG

_NEG_INF = float("-inf")
_POS_INF = float("inf")


def _sig(x):
    return 1.0 / (1.0 + jnp.exp(-x))


def _select_body(N, W, B, pts_ref, ptsT_ref, gp_ref, idx_ref, rad_ref):
    px = pts_ref[:, 0:1]
    py = pts_ref[:, 1:2]
    pz = pts_ref[:, 2:3]
    qx = ptsT_ref[0:1, :]
    qy = ptsT_ref[1:2, :]
    qz = ptsT_ref[2:3, :]
    dx = px - qx
    dy = py - qy
    dz = pz - qz
    d2 = (dx * dx + dy * dy) + dz * dz
    dist = jnp.sqrt(d2 + 1e-12)
    iota = lax.broadcasted_iota(jnp.int32, (B, N), 1)

    work = dist
    cnt = jnp.zeros((B, N), jnp.float32)
    for k in range(KNN):
        mv = jnp.min(work, axis=1, keepdims=True)
        idxk = jnp.min(jnp.where(work == mv, iota, N), axis=1, keepdims=True)
        rad_ref[:, k:k + 1] = mv * mv - 1e-12
        idx_ref[:, k:k + 1] = idxk
        work = jnp.where(iota == idxk, _POS_INF, work)
        cnt = cnt + (iota >= idxk).astype(jnp.float32)
    isknn = work == _POS_INF

    base = jnp.log(1.0 / (dist * dist * dist))
    gexp = jnp.zeros((B, N), jnp.float32)
    for cc in range(KNN + 1):
        sl = gp_ref[:, KNN + 1 - cc: KNN + 1 - cc + N]
        gexp = jnp.where(cnt == cc, sl, gexp)
    score = jnp.where(isknn, _NEG_INF, base + gexp)

    for k in range(SAMPLE):
        mv = jnp.max(score, axis=1, keepdims=True)
        idxk = jnp.min(jnp.where(score == mv, iota, N), axis=1, keepdims=True)
        sel = iota == idxk
        rad_ref[:, KNN + k:KNN + k + 1] = jnp.sum(
            jnp.where(sel, d2, 0.0), axis=1, keepdims=True)
        idx_ref[:, KNN + k:KNN + k + 1] = idxk
        score = jnp.where(sel, _NEG_INF, score)

    idx_ref[:, DEG:] = jnp.zeros((B, 64 - DEG), jnp.int32)
    rad_ref[:, DEG:] = jnp.zeros((B, 64 - DEG), jnp.float32)


def _select(pts, ptsT, gp, N, W, B=128):
    grid = N // B
    fn = pl.pallas_call(
        functools.partial(_select_body, N, W, B),
        grid=(grid,),
        in_specs=[
            pl.BlockSpec((B, 8), lambda i: (i, 0)),
            pl.BlockSpec((8, N), lambda i: (0, 0)),
            pl.BlockSpec((B, W), lambda i: (i, 0)),
        ],
        out_specs=[
            pl.BlockSpec((B, 64), lambda i: (i, 0)),
            pl.BlockSpec((B, 64), lambda i: (i, 0)),
        ],
        out_shape=[
            jax.ShapeDtypeStruct((N, 64), jnp.int32),
            jax.ShapeDtypeStruct((N, 64), jnp.float32),
        ],
    )
    return fn(pts, ptsT, gp)


def _proj_body(h_ref, wst_ref, wdt_ref, bias_ref, hs_ref, hd_ref):
    hh = h_ref[...]
    hs_ref[...] = jnp.dot(hh, wst_ref[...],
                          preferred_element_type=jnp.float32) + bias_ref[0:1, :]
    hd = jnp.dot(hh, wdt_ref[...], preferred_element_type=jnp.float32)
    lo = lax.bitcast_convert_type(hd[:, :D // 2].astype(jnp.bfloat16),
                                  jnp.uint16).astype(jnp.uint32)
    hi = lax.bitcast_convert_type(hd[:, D // 2:].astype(jnp.bfloat16),
                                  jnp.uint16).astype(jnp.uint32)
    hd_ref[...] = lax.bitcast_convert_type((hi << 16) | lo, jnp.float32)


def _proj(h, wst, wdt, biasS):
    fn = pl.pallas_call(
        _proj_body,
        out_shape=[
            jax.ShapeDtypeStruct((N_TOT, D), jnp.float32),
            jax.ShapeDtypeStruct((N_TOT, D // 2), jnp.float32),
        ],
    )
    return fn(h, wst, wdt, biasS)


_SC_WORKERS = 32
_SC_CHUNK = 120
_SC_PER_W = E_TOT // _SC_WORKERS
_SC_NCHUNK = _SC_PER_W // _SC_CHUNK


def _sc_gather_body(table_hbm, idx_hbm, out_hbm, idx_v, rows0, rows1, rows2,
                    g0, g1, g2, w0, w1, w2):
    wid = lax.axis_index("s") * 2 + lax.axis_index("c")
    base = wid * _SC_PER_W
    pltpu.sync_copy(idx_hbm.at[pl.ds(base, _SC_PER_W)], idx_v)

    bufs = (rows0, rows1, rows2)
    gsem = (g0, g1, g2)
    wsem = (w0, w1, w2)

    def gather(c, b):
        return pltpu.async_copy(
            table_hbm.at[idx_v.at[pl.ds(c * _SC_CHUNK, _SC_CHUNK)]],
            bufs[b], gsem[b])

    gh = {0: gather(0, 0), 1: gather(1, 1)}
    wh = {}
    for c in range(_SC_NCHUNK):
        b = c % 3
        gh[c].wait()
        wh[c] = pltpu.async_copy(
            bufs[b], out_hbm.at[pl.ds(base + c * _SC_CHUNK, _SC_CHUNK)],
            wsem[b])
        if c + 2 < _SC_NCHUNK:
            if c >= 1:
                wh[c - 1].wait()
            gh[c + 2] = gather(c + 2, (c + 2) % 3)
    for c in range(_SC_NCHUNK - 3, _SC_NCHUNK):
        wh[c].wait()


def _sc_gather(table, idx):
    mesh = plsc.VectorSubcoreMesh(core_axis_name="c", subcore_axis_name="s")
    fn = pl.kernel(
        _sc_gather_body,
        out_type=jax.ShapeDtypeStruct((E_TOT, D // 2), jnp.float32),
        mesh=mesh,
        scratch_types=[
            pltpu.VMEM((_SC_PER_W,), jnp.int32),
            pltpu.VMEM((_SC_CHUNK, D // 2), jnp.float32),
            pltpu.VMEM((_SC_CHUNK, D // 2), jnp.float32),
            pltpu.VMEM((_SC_CHUNK, D // 2), jnp.float32),
            pltpu.SemaphoreType.DMA,
            pltpu.SemaphoreType.DMA,
            pltpu.SemaphoreType.DMA,
            pltpu.SemaphoreType.DMA,
            pltpu.SemaphoreType.DMA,
            pltpu.SemaphoreType.DMA,
        ],
    )
    return fn(table, idx)


_NB = 32
_EB = _NB * DEG


def _edge_body(hdg_ref, hs_ref, h_ref, rad_ref, we2t_ref,
               wn1ht_ref, wn1at_ref, wn2t_ref, wstn_ref, wdtn_ref, vec_ref,
               out_ref, hsn_ref, hdn_ref):
    f32 = jnp.float32
    row_node = lax.broadcasted_iota(jnp.int32, (_EB, _NB), 0) // DEG
    col16 = lax.broadcasted_iota(jnp.int32, (_EB, _NB), 1)
    A = (row_node == col16).astype(f32)
    rowm = lax.broadcasted_iota(jnp.int32, (_EB, DEG), 0) % DEG
    colm = lax.broadcasted_iota(jnp.int32, (_EB, DEG), 1)
    C = (rowm == colm).astype(f32)
    srow = lax.broadcasted_iota(jnp.int32, (_NB, _EB), 0)
    scol = lax.broadcasted_iota(jnp.int32, (_NB, _EB), 1) // DEG
    S = (srow == scol).astype(f32)

    be2 = vec_ref[0:1, :]
    bn1 = vec_ref[1:2, :]
    bn2 = vec_ref[2:3, :]
    ba = vec_ref[3:4, 0:1]
    wr = vec_ref[4:5, :]
    wa = vec_ref[5:6, :]

    radB = rad_ref[:, :DEG]
    t1 = jnp.dot(A, radB, preferred_element_type=f32)
    rflat = jnp.sum(t1 * C, axis=1, keepdims=True)

    w = lax.bitcast_convert_type(hdg_ref[...], jnp.uint32)
    lo = lax.bitcast_convert_type((w & 0xFFFF).astype(jnp.uint16),
                                  jnp.bfloat16)
    hi = lax.bitcast_convert_type((w >> 16).astype(jnp.uint16), jnp.bfloat16)
    hdg = jnp.concatenate([lo, hi], axis=1)
    hsE = jnp.dot(A, hs_ref[...], preferred_element_type=f32)
    pre = hsE + hdg.astype(f32) + rflat * wr
    m1 = pre * _sig(pre)
    t = jnp.dot(m1.astype(jnp.bfloat16), we2t_ref[...],
                preferred_element_type=f32) + be2
    m2 = t * _sig(t)
    gate = _sig(jnp.sum(m2 * wa, axis=1, keepdims=True) + ba)
    m3 = m2 * gate
    agg = jnp.dot(S, m3, preferred_element_type=f32)

    hB = h_ref[...]
    t2 = (jnp.dot(hB, wn1ht_ref[...], preferred_element_type=f32)
          + jnp.dot(agg, wn1at_ref[...], preferred_element_type=f32) + bn1)
    o1 = t2 * _sig(t2)
    hnew = hB + jnp.dot(o1, wn2t_ref[...], preferred_element_type=f32) + bn2
    out_ref[...] = hnew

    be1n = vec_ref[6:7, :]
    hsn_ref[...] = jnp.dot(hnew, wstn_ref[...],
                           preferred_element_type=f32) + be1n
    hdn = jnp.dot(hnew, wdtn_ref[...], preferred_element_type=f32)
    lo2 = lax.bitcast_convert_type(hdn[:, :D // 2].astype(jnp.bfloat16),
                                   jnp.uint16).astype(jnp.uint32)
    hi2 = lax.bitcast_convert_type(hdn[:, D // 2:].astype(jnp.bfloat16),
                                   jnp.uint16).astype(jnp.uint32)
    hdn_ref[...] = lax.bitcast_convert_type((hi2 << 16) | lo2, jnp.float32)


def _edge(hdg, hs, h, radial, we2t, wn1ht, wn1at, wn2t, wstn, wdtn, vecS):
    grid = N_TOT // _NB
    fn = pl.pallas_call(
        _edge_body,
        grid=(grid,),
        in_specs=[
            pl.BlockSpec((_EB, D // 2), lambda i: (i, 0)),
            pl.BlockSpec((_NB, D), lambda i: (i, 0)),
            pl.BlockSpec((_NB, D), lambda i: (i, 0)),
            pl.BlockSpec((_NB, 64), lambda i: (i, 0)),
            pl.BlockSpec((D, D), lambda i: (0, 0)),
            pl.BlockSpec((D, D), lambda i: (0, 0)),
            pl.BlockSpec((D, D), lambda i: (0, 0)),
            pl.BlockSpec((D, D), lambda i: (0, 0)),
            pl.BlockSpec((D, D), lambda i: (0, 0)),
            pl.BlockSpec((D, D), lambda i: (0, 0)),
            pl.BlockSpec((8, D), lambda i: (0, 0)),
        ],
        out_specs=[
            pl.BlockSpec((_NB, D), lambda i: (i, 0)),
            pl.BlockSpec((_NB, D), lambda i: (i, 0)),
            pl.BlockSpec((_NB, D // 2), lambda i: (i, 0)),
        ],
        out_shape=[
            jax.ShapeDtypeStruct((N_TOT, D), jnp.float32),
            jax.ShapeDtypeStruct((N_TOT, D), jnp.float32),
            jax.ShapeDtypeStruct((N_TOT, D // 2), jnp.float32),
        ],
    )
    return fn(hdg, hs, h, radial, we2t, wn1ht, wn1at, wn2t, wstn, wdtn, vecS)


def _ln(x, g, b):
    mu = jnp.mean(x, axis=1, keepdims=True)
    var = jnp.mean((x - mu) ** 2, axis=1, keepdims=True)
    return (x - mu) / jnp.sqrt(var + 1e-5) * g + b


def _post_body(h_ref, x_ref, wqt_ref, wkt_ref, wvt_ref, wot_ref,
               wf1t_ref, wf2t_ref, bias_ref, bf1_ref, ln_ref, out_ref):
    f32 = jnp.float32
    nh, hd = 8, 32
    inv_s = 1.0 / (hd ** 0.5)

    h0 = h_ref[...] + x_ref[...]
    h1 = _ln(h0, ln_ref[0:1, :], ln_ref[1:2, :])

    q = jnp.dot(h1, wqt_ref[...], preferred_element_type=f32) + bias_ref[0:1, :]
    kk = jnp.dot(h1, wkt_ref[...], preferred_element_type=f32) + bias_ref[1:2, :]
    v = jnp.dot(h1, wvt_ref[...], preferred_element_type=f32) + bias_ref[2:3, :]

    def attend(qm, km, vm):
        outs = []
        for head in range(nh):
            s0 = head * hd
            qh = qm[:, s0:s0 + hd]
            kh = km[:, s0:s0 + hd]
            vh = vm[:, s0:s0 + hd]
            sc = lax.dot_general(qh, kh, (((1,), (1,)), ((), ())),
                                 preferred_element_type=f32) * inv_s
            sc = sc - jnp.max(sc, axis=1, keepdims=True)
            e = jnp.exp(sc)
            a = e / jnp.sum(e, axis=1, keepdims=True)
            outs.append(jnp.dot(a, vh, preferred_element_type=f32))
        return jnp.concatenate(outs, axis=1)

    att_rec = attend(q[:N_REC], kk[N_REC:], v[N_REC:])
    att_lig = attend(q[N_REC:], kk[:N_REC], v[:N_REC])
    att = jnp.concatenate([att_rec, att_lig], axis=0)
    att = jnp.dot(att, wot_ref[...], preferred_element_type=f32) + bias_ref[3:4, :]

    h2 = _ln(att + h1, ln_ref[2:3, :], ln_ref[3:4, :])
    f = jnp.dot(h2, wf1t_ref[...], preferred_element_type=f32) + bf1_ref[0:1, :]
    f = f * _sig(f)
    f2 = jnp.dot(f, wf2t_ref[...], preferred_element_type=f32) + bias_ref[4:5, :]
    out_ref[...] = _ln(f2 + h2, ln_ref[4:5, :], ln_ref[5:6, :])


def _post(h, xin, wqt, wkt, wvt, wot, wf1t, wf2t, biasP, bf1, lnS):
    fn = pl.pallas_call(
        _post_body,
        out_shape=jax.ShapeDtypeStruct((N_TOT, D), jnp.float32),
    )
    return fn(h, xin, wqt, wkt, wvt, wot, wf1t, wf2t, biasP, bf1, lnS)


def _gumbel_table(key, n):
    u = jax.random.uniform(key, (n, n - KNN), minval=1e-9, maxval=1.0)
    g = -jnp.log(-jnp.log(u))
    w = ((KNN + 1 + (n - KNN) + 127) // 128 + 1) * 128
    gp = jnp.zeros((n, w), jnp.float32)
    return gp.at[:, KNN + 1:KNN + 1 + (n - KNN)].set(g), w


def _pts_forms(pos):
    p = pos[:, 1]
    pts = jnp.pad(p, ((0, 0), (0, 5)))
    return pts, pts.T.reshape(8, -1)


def kernel(rec_x, lig_x, rec_pos, lig_pos, params):
    p = params
    k1, k2 = jax.random.split(jax.random.key(42))
    gp_rec, w_rec = _gumbel_table(k1, N_REC)
    gp_lig, w_lig = _gumbel_table(k2, N_LIG)

    pts_r, ptsT_r = _pts_forms(rec_pos)
    pts_l, ptsT_l = _pts_forms(lig_pos)

    idx_r, rad_r = _select(pts_r, ptsT_r, gp_rec, N_REC, w_rec)
    idx_l, rad_l = _select(pts_l, ptsT_l, gp_lig, N_LIG, w_lig)

    idx_all = jnp.concatenate([
        idx_r[:, :DEG].reshape(-1),
        idx_l[:, :DEG].reshape(-1) + N_REC,
    ]).astype(jnp.int32)
    radial = jnp.concatenate([rad_r, rad_l], axis=0)

    h = jnp.concatenate([rec_x, lig_x], axis=0)
    layers = p["egnn"]
    wst0 = layers[0]["We1"][:, :D].T
    wdt0 = layers[0]["We1"][:, D:2 * D].T
    projB = jnp.zeros((8, D), jnp.float32).at[0].set(layers[0]["be1"])
    hs, hd_packed = _proj(h, wst0, wdt0, projB)
    for li, lp in enumerate(layers):
        nxt = layers[li + 1] if li + 1 < len(layers) else layers[0]
        vecS = (jnp.zeros((8, D), jnp.float32)
                .at[0].set(lp["be2"])
                .at[1].set(lp["bn1"])
                .at[2].set(lp["bn2"])
                .at[3].set(lp["ba"][0])
                .at[4].set(lp["We1"][:, 2 * D])
                .at[5].set(lp["Wa"][0])
                .at[6].set(nxt["be1"]))
        hdg = _sc_gather(hd_packed, idx_all)
        h, hs, hd_packed = _edge(
            hdg, hs, h, radial, lp["We2"].T.astype(jnp.bfloat16),
            lp["Wn1"][:, :D].T, lp["Wn1"][:, D:].T, lp["Wn2"].T,
            nxt["We1"][:, :D].T, nxt["We1"][:, D:2 * D].T, vecS)

    xin = jnp.concatenate([rec_x, lig_x], axis=0)
    biasP = (jnp.zeros((8, D), jnp.float32)
             .at[0].set(p["bin"][:D])
             .at[1].set(p["bin"][D:2 * D])
             .at[2].set(p["bin"][2 * D:])
             .at[3].set(p["bout"])
             .at[4].set(p["bf2"]))
    bf1 = jnp.zeros((8, 2 * D), jnp.float32).at[0].set(p["bf1"])
    lnS = (jnp.zeros((8, D), jnp.float32)
           .at[0].set(p["n1g"]).at[1].set(p["n1b"])
           .at[2].set(p["n2g"]).at[3].set(p["n2b"])
           .at[4].set(p["n3g"]).at[5].set(p["n3b"]))
    out = _post(h, xin,
                p["Win"][:D].T, p["Win"][D:2 * D].T, p["Win"][2 * D:].T,
                p["Wout"].T, p["Wf1"].T, p["Wf2"].T, biasP, bf1, lnS)
    return out


# speedup vs baseline: 2.8150x; 1.0571x over previous
import functools

import jax
import jax.numpy as jnp
from jax import lax
from jax.experimental import pallas as pl
from jax.experimental.pallas import tpu as pltpu
from jax.experimental.pallas import tpu_sc as plsc

D = 256
KNN = 20
SAMPLE = 40
DEG = KNN + SAMPLE
N_REC = 1024
N_LIG = 256
N_TOT = N_REC + N_LIG
E_TOT = N_TOT * DEG

_NEG_INF = float("-inf")
_POS_INF = float("inf")


def _sig(x):
    return 1.0 / (1.0 + jnp.exp(-x))


def _select_body(N, W, B, pts_ref, ptsT_ref, gp_ref, idx_ref, rad_ref):
    px = pts_ref[:, 0:1]
    py = pts_ref[:, 1:2]
    pz = pts_ref[:, 2:3]
    qx = ptsT_ref[0:1, :]
    qy = ptsT_ref[1:2, :]
    qz = ptsT_ref[2:3, :]
    dx = px - qx
    dy = py - qy
    dz = pz - qz
    d2 = (dx * dx + dy * dy) + dz * dz
    dist = jnp.sqrt(d2 + 1e-12)
    iota = lax.broadcasted_iota(jnp.int32, (B, N), 1)

    work = dist
    cnt = jnp.zeros((B, N), jnp.float32)
    for k in range(KNN):
        mv = jnp.min(work, axis=1, keepdims=True)
        idxk = jnp.min(jnp.where(work == mv, iota, N), axis=1, keepdims=True)
        rad_ref[:, k:k + 1] = mv * mv - 1e-12
        idx_ref[:, k:k + 1] = idxk
        work = jnp.where(iota == idxk, _POS_INF, work)
        cnt = cnt + (iota >= idxk).astype(jnp.float32)
    isknn = work == _POS_INF

    base = jnp.log(1.0 / (dist * dist * dist))
    gexp = jnp.zeros((B, N), jnp.float32)
    for cc in range(KNN + 1):
        sl = gp_ref[:, KNN + 1 - cc: KNN + 1 - cc + N]
        gexp = jnp.where(cnt == cc, sl, gexp)
    score = jnp.where(isknn, _NEG_INF, base + gexp)

    for k in range(SAMPLE):
        mv = jnp.max(score, axis=1, keepdims=True)
        idxk = jnp.min(jnp.where(score == mv, iota, N), axis=1, keepdims=True)
        sel = iota == idxk
        rad_ref[:, KNN + k:KNN + k + 1] = jnp.sum(
            jnp.where(sel, d2, 0.0), axis=1, keepdims=True)
        idx_ref[:, KNN + k:KNN + k + 1] = idxk
        score = jnp.where(sel, _NEG_INF, score)

    idx_ref[:, DEG:] = jnp.zeros((B, 64 - DEG), jnp.int32)
    rad_ref[:, DEG:] = jnp.zeros((B, 64 - DEG), jnp.float32)


def _select(pts, ptsT, gp, N, W, B=256):
    grid = N // B
    fn = pl.pallas_call(
        functools.partial(_select_body, N, W, B),
        grid=(grid,),
        in_specs=[
            pl.BlockSpec((B, 8), lambda i: (i, 0)),
            pl.BlockSpec((8, N), lambda i: (0, 0)),
            pl.BlockSpec((B, W), lambda i: (i, 0)),
        ],
        out_specs=[
            pl.BlockSpec((B, 64), lambda i: (i, 0)),
            pl.BlockSpec((B, 64), lambda i: (i, 0)),
        ],
        out_shape=[
            jax.ShapeDtypeStruct((N, 64), jnp.int32),
            jax.ShapeDtypeStruct((N, 64), jnp.float32),
        ],
    )
    return fn(pts, ptsT, gp)


def _proj_body(h_ref, wst_ref, wdt_ref, bias_ref, hs_ref, hd_ref):
    hh = h_ref[...]
    hs_ref[...] = jnp.dot(hh, wst_ref[...],
                          preferred_element_type=jnp.float32) + bias_ref[0:1, :]
    hd = jnp.dot(hh, wdt_ref[...], preferred_element_type=jnp.float32)
    lo = lax.bitcast_convert_type(hd[:, :D // 2].astype(jnp.bfloat16),
                                  jnp.uint16).astype(jnp.uint32)
    hi = lax.bitcast_convert_type(hd[:, D // 2:].astype(jnp.bfloat16),
                                  jnp.uint16).astype(jnp.uint32)
    hd_ref[...] = lax.bitcast_convert_type((hi << 16) | lo, jnp.float32)


def _proj(h, wst, wdt, biasS):
    fn = pl.pallas_call(
        _proj_body,
        out_shape=[
            jax.ShapeDtypeStruct((N_TOT, D), jnp.float32),
            jax.ShapeDtypeStruct((N_TOT, D // 2), jnp.float32),
        ],
    )
    return fn(h, wst, wdt, biasS)


_SC_WORKERS = 32
_SC_CHUNK = 120
_SC_PER_W = E_TOT // _SC_WORKERS
_SC_NCHUNK = _SC_PER_W // _SC_CHUNK


def _sc_gather_body(table_hbm, idx_hbm, out_hbm, idx_v, rows0, rows1, rows2,
                    g0, g1, g2, w0, w1, w2):
    wid = lax.axis_index("s") * 2 + lax.axis_index("c")
    base = wid * _SC_PER_W
    pltpu.sync_copy(idx_hbm.at[pl.ds(base, _SC_PER_W)], idx_v)

    bufs = (rows0, rows1, rows2)
    gsem = (g0, g1, g2)
    wsem = (w0, w1, w2)

    def gather(c, b):
        return pltpu.async_copy(
            table_hbm.at[idx_v.at[pl.ds(c * _SC_CHUNK, _SC_CHUNK)]],
            bufs[b], gsem[b])

    gh = {0: gather(0, 0), 1: gather(1, 1)}
    wh = {}
    for c in range(_SC_NCHUNK):
        b = c % 3
        gh[c].wait()
        wh[c] = pltpu.async_copy(
            bufs[b], out_hbm.at[pl.ds(base + c * _SC_CHUNK, _SC_CHUNK)],
            wsem[b])
        if c + 2 < _SC_NCHUNK:
            if c >= 1:
                wh[c - 1].wait()
            gh[c + 2] = gather(c + 2, (c + 2) % 3)
    for c in range(_SC_NCHUNK - 3, _SC_NCHUNK):
        wh[c].wait()


def _sc_gather(table, idx):
    mesh = plsc.VectorSubcoreMesh(core_axis_name="c", subcore_axis_name="s")
    fn = pl.kernel(
        _sc_gather_body,
        out_type=jax.ShapeDtypeStruct((E_TOT, D // 2), jnp.float32),
        mesh=mesh,
        scratch_types=[
            pltpu.VMEM((_SC_PER_W,), jnp.int32),
            pltpu.VMEM((_SC_CHUNK, D // 2), jnp.float32),
            pltpu.VMEM((_SC_CHUNK, D // 2), jnp.float32),
            pltpu.VMEM((_SC_CHUNK, D // 2), jnp.float32),
            pltpu.SemaphoreType.DMA,
            pltpu.SemaphoreType.DMA,
            pltpu.SemaphoreType.DMA,
            pltpu.SemaphoreType.DMA,
            pltpu.SemaphoreType.DMA,
            pltpu.SemaphoreType.DMA,
        ],
    )
    return fn(table, idx)


_NB = 32
_EB = _NB * DEG


def _edge_body(hdg_ref, hs_ref, h_ref, rad_ref, we2t_ref,
               wn1ht_ref, wn1at_ref, wn2t_ref, wstn_ref, wdtn_ref, vec_ref,
               out_ref, hsn_ref, hdn_ref):
    f32 = jnp.float32
    row_node = lax.broadcasted_iota(jnp.int32, (_EB, _NB), 0) // DEG
    col16 = lax.broadcasted_iota(jnp.int32, (_EB, _NB), 1)
    A = (row_node == col16).astype(f32)
    rowm = lax.broadcasted_iota(jnp.int32, (_EB, DEG), 0) % DEG
    colm = lax.broadcasted_iota(jnp.int32, (_EB, DEG), 1)
    C = (rowm == colm).astype(f32)
    srow = lax.broadcasted_iota(jnp.int32, (_NB, _EB), 0)
    scol = lax.broadcasted_iota(jnp.int32, (_NB, _EB), 1) // DEG
    S = (srow == scol).astype(f32)

    be2 = vec_ref[0:1, :]
    bn1 = vec_ref[1:2, :]
    bn2 = vec_ref[2:3, :]
    ba = vec_ref[3:4, 0:1]
    wr = vec_ref[4:5, :]
    wa = vec_ref[5:6, :]

    radB = rad_ref[:, :DEG]
    t1 = jnp.dot(A, radB, preferred_element_type=f32)
    rflat = jnp.sum(t1 * C, axis=1, keepdims=True)

    w = lax.bitcast_convert_type(hdg_ref[...], jnp.uint32)
    lo = lax.bitcast_convert_type(w << 16, f32)
    hi = lax.bitcast_convert_type(w & jnp.uint32(0xFFFF0000), f32)
    hdg = jnp.concatenate([lo, hi], axis=1)
    hsE = jnp.dot(A, hs_ref[...], preferred_element_type=f32)
    pre = hsE + hdg + rflat * wr
    m1 = pre * _sig(pre)
    t = jnp.dot(m1.astype(jnp.bfloat16), we2t_ref[...],
                preferred_element_type=f32) + be2
    m2 = t * _sig(t)
    gate = _sig(jnp.sum(m2 * wa, axis=1, keepdims=True) + ba)
    m3 = m2 * gate
    agg = jnp.dot(S, m3, preferred_element_type=f32)

    hB = h_ref[...]
    t2 = (jnp.dot(hB, wn1ht_ref[...], preferred_element_type=f32)
          + jnp.dot(agg, wn1at_ref[...], preferred_element_type=f32) + bn1)
    o1 = t2 * _sig(t2)
    hnew = hB + jnp.dot(o1, wn2t_ref[...], preferred_element_type=f32) + bn2
    out_ref[...] = hnew

    be1n = vec_ref[6:7, :]
    hsn_ref[...] = jnp.dot(hnew, wstn_ref[...],
                           preferred_element_type=f32) + be1n
    hdn = jnp.dot(hnew, wdtn_ref[...], preferred_element_type=f32)
    lo2 = lax.bitcast_convert_type(hdn[:, :D // 2].astype(jnp.bfloat16),
                                   jnp.uint16).astype(jnp.uint32)
    hi2 = lax.bitcast_convert_type(hdn[:, D // 2:].astype(jnp.bfloat16),
                                   jnp.uint16).astype(jnp.uint32)
    hdn_ref[...] = lax.bitcast_convert_type((hi2 << 16) | lo2, jnp.float32)


def _edge(hdg, hs, h, radial, we2t, wn1ht, wn1at, wn2t, wstn, wdtn, vecS):
    grid = N_TOT // _NB
    fn = pl.pallas_call(
        _edge_body,
        grid=(grid,),
        in_specs=[
            pl.BlockSpec((_EB, D // 2), lambda i: (i, 0)),
            pl.BlockSpec((_NB, D), lambda i: (i, 0)),
            pl.BlockSpec((_NB, D), lambda i: (i, 0)),
            pl.BlockSpec((_NB, 64), lambda i: (i, 0)),
            pl.BlockSpec((D, D), lambda i: (0, 0)),
            pl.BlockSpec((D, D), lambda i: (0, 0)),
            pl.BlockSpec((D, D), lambda i: (0, 0)),
            pl.BlockSpec((D, D), lambda i: (0, 0)),
            pl.BlockSpec((D, D), lambda i: (0, 0)),
            pl.BlockSpec((D, D), lambda i: (0, 0)),
            pl.BlockSpec((8, D), lambda i: (0, 0)),
        ],
        out_specs=[
            pl.BlockSpec((_NB, D), lambda i: (i, 0)),
            pl.BlockSpec((_NB, D), lambda i: (i, 0)),
            pl.BlockSpec((_NB, D // 2), lambda i: (i, 0)),
        ],
        out_shape=[
            jax.ShapeDtypeStruct((N_TOT, D), jnp.float32),
            jax.ShapeDtypeStruct((N_TOT, D), jnp.float32),
            jax.ShapeDtypeStruct((N_TOT, D // 2), jnp.float32),
        ],
    )
    return fn(hdg, hs, h, radial, we2t, wn1ht, wn1at, wn2t, wstn, wdtn, vecS)


def _ln(x, g, b):
    mu = jnp.mean(x, axis=1, keepdims=True)
    var = jnp.mean((x - mu) ** 2, axis=1, keepdims=True)
    return (x - mu) / jnp.sqrt(var + 1e-5) * g + b


def _post_body(h_ref, x_ref, wqt_ref, wkt_ref, wvt_ref, wot_ref,
               wf1t_ref, wf2t_ref, bias_ref, bf1_ref, ln_ref, out_ref):
    f32 = jnp.float32
    nh, hd = 8, 32
    inv_s = 1.0 / (hd ** 0.5)

    h0 = h_ref[...] + x_ref[...]
    h1 = _ln(h0, ln_ref[0:1, :], ln_ref[1:2, :])

    q = jnp.dot(h1, wqt_ref[...], preferred_element_type=f32) + bias_ref[0:1, :]
    kk = jnp.dot(h1, wkt_ref[...], preferred_element_type=f32) + bias_ref[1:2, :]
    v = jnp.dot(h1, wvt_ref[...], preferred_element_type=f32) + bias_ref[2:3, :]

    def attend(qm, km, vm):
        outs = []
        for head in range(nh):
            s0 = head * hd
            qh = qm[:, s0:s0 + hd]
            kh = km[:, s0:s0 + hd]
            vh = vm[:, s0:s0 + hd]
            sc = lax.dot_general(qh, kh, (((1,), (1,)), ((), ())),
                                 preferred_element_type=f32) * inv_s
            sc = sc - jnp.max(sc, axis=1, keepdims=True)
            e = jnp.exp(sc)
            a = e / jnp.sum(e, axis=1, keepdims=True)
            outs.append(jnp.dot(a, vh, preferred_element_type=f32))
        return jnp.concatenate(outs, axis=1)

    att_rec = attend(q[:N_REC], kk[N_REC:], v[N_REC:])
    att_lig = attend(q[N_REC:], kk[:N_REC], v[:N_REC])
    att = jnp.concatenate([att_rec, att_lig], axis=0)
    att = jnp.dot(att, wot_ref[...], preferred_element_type=f32) + bias_ref[3:4, :]

    h2 = _ln(att + h1, ln_ref[2:3, :], ln_ref[3:4, :])
    f = jnp.dot(h2, wf1t_ref[...], preferred_element_type=f32) + bf1_ref[0:1, :]
    f = f * _sig(f)
    f2 = jnp.dot(f, wf2t_ref[...], preferred_element_type=f32) + bias_ref[4:5, :]
    out_ref[...] = _ln(f2 + h2, ln_ref[4:5, :], ln_ref[5:6, :])


def _post(h, xin, wqt, wkt, wvt, wot, wf1t, wf2t, biasP, bf1, lnS):
    fn = pl.pallas_call(
        _post_body,
        out_shape=jax.ShapeDtypeStruct((N_TOT, D), jnp.float32),
    )
    return fn(h, xin, wqt, wkt, wvt, wot, wf1t, wf2t, biasP, bf1, lnS)


def _gumbel_table(key, n):
    u = jax.random.uniform(key, (n, n - KNN), minval=1e-9, maxval=1.0)
    g = -jnp.log(-jnp.log(u))
    w = ((KNN + 1 + (n - KNN) + 127) // 128 + 1) * 128
    gp = jnp.zeros((n, w), jnp.float32)
    return gp.at[:, KNN + 1:KNN + 1 + (n - KNN)].set(g), w


def _pts_forms(pos):
    p = pos[:, 1]
    pts = jnp.pad(p, ((0, 0), (0, 5)))
    return pts, pts.T.reshape(8, -1)


def kernel(rec_x, lig_x, rec_pos, lig_pos, params):
    p = params
    k1, k2 = jax.random.split(jax.random.key(42))
    gp_rec, w_rec = _gumbel_table(k1, N_REC)
    gp_lig, w_lig = _gumbel_table(k2, N_LIG)

    pts_r, ptsT_r = _pts_forms(rec_pos)
    pts_l, ptsT_l = _pts_forms(lig_pos)

    idx_r, rad_r = _select(pts_r, ptsT_r, gp_rec, N_REC, w_rec)
    idx_l, rad_l = _select(pts_l, ptsT_l, gp_lig, N_LIG, w_lig)

    idx_all = jnp.concatenate([
        idx_r[:, :DEG].reshape(-1),
        idx_l[:, :DEG].reshape(-1) + N_REC,
    ]).astype(jnp.int32)
    radial = jnp.concatenate([rad_r, rad_l], axis=0)

    h = jnp.concatenate([rec_x, lig_x], axis=0)
    layers = p["egnn"]
    wst0 = layers[0]["We1"][:, :D].T
    wdt0 = layers[0]["We1"][:, D:2 * D].T
    projB = jnp.zeros((8, D), jnp.float32).at[0].set(layers[0]["be1"])
    hs, hd_packed = _proj(h, wst0, wdt0, projB)
    for li, lp in enumerate(layers):
        nxt = layers[li + 1] if li + 1 < len(layers) else layers[0]
        vecS = (jnp.zeros((8, D), jnp.float32)
                .at[0].set(lp["be2"])
                .at[1].set(lp["bn1"])
                .at[2].set(lp["bn2"])
                .at[3].set(lp["ba"][0])
                .at[4].set(lp["We1"][:, 2 * D])
                .at[5].set(lp["Wa"][0])
                .at[6].set(nxt["be1"]))
        hdg = _sc_gather(hd_packed, idx_all)
        h, hs, hd_packed = _edge(
            hdg, hs, h, radial, lp["We2"].T.astype(jnp.bfloat16),
            lp["Wn1"][:, :D].T, lp["Wn1"][:, D:].T, lp["Wn2"].T,
            nxt["We1"][:, :D].T, nxt["We1"][:, D:2 * D].T, vecS)

    xin = jnp.concatenate([rec_x, lig_x], axis=0)
    biasP = (jnp.zeros((8, D), jnp.float32)
             .at[0].set(p["bin"][:D])
             .at[1].set(p["bin"][D:2 * D])
             .at[2].set(p["bin"][2 * D:])
             .at[3].set(p["bout"])
             .at[4].set(p["bf2"]))
    bf1 = jnp.zeros((8, 2 * D), jnp.float32).at[0].set(p["bf1"])
    lnS = (jnp.zeros((8, D), jnp.float32)
           .at[0].set(p["n1g"]).at[1].set(p["n1b"])
           .at[2].set(p["n2g"]).at[3].set(p["n2b"])
           .at[4].set(p["n3g"]).at[5].set(p["n3b"]))
    out = _post(h, xin,
                p["Win"][:D].T, p["Win"][D:2 * D].T, p["Win"][2 * D:].T,
                p["Wout"].T, p["Wf1"].T, p["Wf2"].T, biasP, bf1, lnS)
    return out
